# Initial kernel scaffold; baseline (speedup 1.0000x reference)
import jax
import jax.numpy as jnp
from jax.experimental import pallas as pl


def kernel(x, edge_index, W, b, ln_gamma, ln_beta):
    raise NotImplementedError("write your pallas kernel here")


# speedup vs baseline: 8.4819x; 8.4819x over previous
import functools

import jax
import jax.numpy as jnp
from jax import lax
from jax.experimental import pallas as pl
from jax.experimental.pallas import tpu as pltpu
from jax.experimental.pallas import tpu_sc as plsc

NC = 2
NS = 16
CH = 128


def _deg_body(np_, ca, dst_hbm, ones_hbm, zeros_hbm, out_hbm, idx_v, ones_v, acc_sh):
    c = lax.axis_index("c")
    s = lax.axis_index("s")
    wid = s * NC + c
    rt = np_ // NS
    pltpu.sync_copy(zeros_hbm.at[pl.ds(s * rt, rt)], acc_sh.at[pl.ds(s * rt, rt)])
    pltpu.sync_copy(ones_hbm, ones_v)
    pltpu.sync_copy(dst_hbm.at[wid], idx_v)
    plsc.subcore_barrier()

    @pl.loop(0, ca)
    def _(j):
        pltpu.sync_copy(ones_v, acc_sh.at[idx_v.at[j]], add=True)

    plsc.subcore_barrier()
    pltpu.sync_copy(acc_sh.at[pl.ds(s * rt, rt)], out_hbm.at[c, pl.ds(s * rt, rt)])


def _agg_body(np_, cb, y0_hbm, y1_hbm, src_hbm, dst_hbm, zeros_hbm, out_hbm,
              src_v, dst_v, buf0, acc_sh, sem0):
    c = lax.axis_index("c")
    s = lax.axis_index("s")
    rt = np_ // NS
    pltpu.sync_copy(zeros_hbm.at[pl.ds(s * rt, rt)], acc_sh.at[pl.ds(s * rt, rt)])
    pltpu.sync_copy(src_hbm.at[s], src_v)
    pltpu.sync_copy(dst_hbm.at[s], dst_v)
    plsc.subcore_barrier()

    def run(y_hbm):
        @pl.loop(0, cb)
        def _(j):
            pltpu.async_copy(y_hbm.at[src_v.at[j]], buf0, sem0).wait()
            pltpu.sync_copy(buf0, acc_sh.at[dst_v.at[j]], add=True)

    @pl.when(c == 0)
    def _():
        run(y0_hbm)

    @pl.when(c == 1)
    def _():
        run(y1_hbm)

    plsc.subcore_barrier()
    pltpu.sync_copy(acc_sh.at[pl.ds(s * rt, rt)], out_hbm.at[c, pl.ds(s * rt, rt)])


def _mm_body(x_ref, w_ref, d0_ref, d1_ref, y_ref):
    xw = lax.dot_general(x_ref[...], w_ref[...], (((1,), (1,)), ((), ())),
                         preferred_element_type=jnp.float32)
    dinv = lax.rsqrt(d0_ref[:, :1] + d1_ref[:, :1] + 1.0)
    y_ref[...] = xw * dinv


def _fin_body(s0_ref, s1_ref, y_ref, d0_ref, d1_ref, b_ref, g_ref, bb_ref, o_ref):
    dinv = lax.rsqrt(d0_ref[:, :1] + d1_ref[:, :1] + 1.0)
    h = jnp.concatenate([s0_ref[0], s1_ref[0]], axis=-1)
    h = (h + y_ref[...]) * dinv + b_ref[...][None, :]
    mean = jnp.mean(h, axis=1, keepdims=True)
    var = jnp.mean((h - mean) ** 2, axis=1, keepdims=True)
    h = (h - mean) * lax.rsqrt(var + 1e-5)
    h = h * g_ref[...][None, :] + bb_ref[...][None, :]
    o_ref[...] = 0.5 * h * (1.0 + lax.erf(h * 0.7071067811865476))


def kernel(x, edge_index, W, b, ln_gamma, ln_beta):
    N, D = x.shape
    DO = W.shape[0]
    E = edge_index.shape[1]
    DH = DO // 2

    NP = ((N + 1 + 2047) // 2048) * 2048
    EP = ((E + 4095) // 4096) * 4096
    CA = EP // (2 * NS * CH)
    CB = EP // (NS * CH)

    src = edge_index[0].astype(jnp.int32)
    dst = edge_index[1].astype(jnp.int32)
    pad = EP - E
    src_p = jnp.concatenate([src, jnp.zeros((pad,), jnp.int32)])
    dst_p = jnp.concatenate([dst, jnp.full((pad,), N, jnp.int32)])
    dst_a = dst_p.reshape(2 * NS, CA, CH)
    src_b = src_p.reshape(NS, CB, CH)
    dst_b = dst_p.reshape(NS, CB, CH)

    ones1 = jnp.ones((CH, DH), jnp.float32)
    zerosb = jnp.zeros((NP, DH), jnp.float32)

    mesh = plsc.VectorSubcoreMesh(core_axis_name="c", subcore_axis_name="s")

    deg_call = functools.partial(pl.kernel,
        out_type=jax.ShapeDtypeStruct((NC, NP, DH), jnp.float32),
        mesh=mesh,
        scratch_types=[
            pltpu.VMEM((CA, CH), jnp.int32),
            pltpu.VMEM((CH, DH), jnp.float32),
            pltpu.VMEM_SHARED((NP, DH), jnp.float32),
        ])(functools.partial(_deg_body, NP, CA))
    deg = deg_call(dst_a, ones1, zerosb)
    deg0 = deg[0]
    deg1 = deg[1]

    R = 400
    grid = (N // R,)
    dspec = pl.BlockSpec((R, DH), lambda i: (i, 0))
    y = pl.pallas_call(
        _mm_body,
        grid=grid,
        in_specs=[
            pl.BlockSpec((R, D), lambda i: (i, 0)),
            pl.BlockSpec((DO, D), lambda i: (0, 0)),
            dspec, dspec,
        ],
        out_specs=pl.BlockSpec((R, DO), lambda i: (i, 0)),
        out_shape=jax.ShapeDtypeStruct((N, DO), jnp.float32),
    )(x, W, deg0, deg1)

    y0 = y[:, :DH]
    y1 = y[:, DH:]

    agg_call = functools.partial(pl.kernel,
        out_type=jax.ShapeDtypeStruct((NC, NP, DH), jnp.float32),
        mesh=mesh,
        scratch_types=[
            pltpu.VMEM((CB, CH), jnp.int32),
            pltpu.VMEM((CB, CH), jnp.int32),
            pltpu.VMEM((CH, DH), jnp.float32),
            pltpu.VMEM_SHARED((NP, DH), jnp.float32),
            pltpu.SemaphoreType.DMA,
        ])(functools.partial(_agg_body, NP, CB))
    s_parts = agg_call(y0, y1, src_b, dst_b, zerosb)

    sspec0 = pl.BlockSpec((1, R, DH), lambda i: (0, i, 0))
    sspec1 = pl.BlockSpec((1, R, DH), lambda i: (1, i, 0))
    vspec = pl.BlockSpec((DO,), lambda i: (0,))
    out = pl.pallas_call(
        _fin_body,
        grid=grid,
        in_specs=[
            sspec0, sspec1,
            pl.BlockSpec((R, DO), lambda i: (i, 0)),
            dspec, dspec,
            vspec, vspec, vspec,
        ],
        out_specs=pl.BlockSpec((R, DO), lambda i: (i, 0)),
        out_shape=jax.ShapeDtypeStruct((N, DO), jnp.float32),
    )(s_parts, s_parts, y, deg0, deg1, b, ln_gamma, ln_beta)
    return out


# speedup vs baseline: 8.8298x; 1.0410x over previous
import functools

import jax
import jax.numpy as jnp
from jax import lax
from jax.experimental import pallas as pl
from jax.experimental.pallas import tpu as pltpu
from jax.experimental.pallas import tpu_sc as plsc

NC = 2
NS = 16
CH = 128


def _deg_body(np_, ca, dst_hbm, ones_hbm, zeros_hbm, out_hbm, idx_v, ones_v, acc_sh):
    c = lax.axis_index("c")
    s = lax.axis_index("s")
    wid = s * NC + c
    rt = np_ // NS
    pltpu.sync_copy(zeros_hbm.at[pl.ds(s * rt, rt)], acc_sh.at[pl.ds(s * rt, rt)])
    pltpu.sync_copy(ones_hbm, ones_v)
    pltpu.sync_copy(dst_hbm.at[wid], idx_v)
    plsc.subcore_barrier()

    @pl.loop(0, ca)
    def _(j):
        pltpu.sync_copy(ones_v, acc_sh.at[idx_v.at[j]], add=True)

    plsc.subcore_barrier()
    pltpu.sync_copy(acc_sh.at[pl.ds(s * rt, rt)], out_hbm.at[c, pl.ds(s * rt, rt)])


def _agg_body(np_, cb, y0_hbm, y1_hbm, src_hbm, dst_hbm, zeros_hbm, out_hbm,
              dst_v, si, buf0, buf1, acc_sh, semS0, semS1, semG0, semG1):
    c = lax.axis_index("c")
    s = lax.axis_index("s")
    rt = np_ // NS
    pltpu.sync_copy(zeros_hbm.at[pl.ds(s * rt, rt)], acc_sh.at[pl.ds(s * rt, rt)])
    pltpu.sync_copy(dst_hbm.at[s], dst_v)
    plsc.subcore_barrier()

    def run(y_hbm):
        pltpu.async_copy(src_hbm.at[s, 0], si.at[0], semS0)
        pltpu.async_copy(src_hbm.at[s, 1], si.at[1], semS1)

        @pl.loop(0, cb, step=2)
        def _(j):
            pltpu.make_async_copy(src_hbm.at[s, j], si.at[0], semS0).wait()
            pltpu.async_copy(y_hbm.at[si.at[0]], buf0, semG0)
            pltpu.make_async_copy(src_hbm.at[s, j + 1], si.at[1], semS1).wait()
            pltpu.async_copy(y_hbm.at[si.at[1]], buf1, semG1)

            pltpu.make_async_copy(y_hbm.at[si.at[0]], buf0, semG0).wait()

            @pl.when(j + 2 < cb)
            def _():
                pltpu.async_copy(src_hbm.at[s, j + 2], si.at[0], semS0)

            pltpu.sync_copy(buf0, acc_sh.at[dst_v.at[j]], add=True)
            pltpu.make_async_copy(y_hbm.at[si.at[1]], buf1, semG1).wait()

            @pl.when(j + 2 < cb)
            def _():
                pltpu.async_copy(src_hbm.at[s, j + 3], si.at[1], semS1)

            pltpu.sync_copy(buf1, acc_sh.at[dst_v.at[j + 1]], add=True)

    @pl.when(c == 0)
    def _():
        run(y0_hbm)

    @pl.when(c == 1)
    def _():
        run(y1_hbm)

    plsc.subcore_barrier()
    pltpu.sync_copy(acc_sh.at[pl.ds(s * rt, rt)], out_hbm.at[c, pl.ds(s * rt, rt)])


def _mm_body(x_ref, w_ref, d0_ref, d1_ref, y0_ref, y1_ref):
    xw = lax.dot_general(x_ref[...], w_ref[...], (((1,), (1,)), ((), ())),
                         preferred_element_type=jnp.float32)
    dinv = lax.rsqrt(d0_ref[:, :1] + d1_ref[:, :1] + 1.0)
    y = xw * dinv
    dh = y.shape[1] // 2
    y0_ref[...] = y[:, :dh]
    y1_ref[...] = y[:, dh:]


def _fin_body(s0_ref, s1_ref, y0_ref, y1_ref, d0_ref, d1_ref, b_ref, g_ref,
              bb_ref, o_ref):
    dinv = lax.rsqrt(d0_ref[:, :1] + d1_ref[:, :1] + 1.0)
    h = jnp.concatenate([s0_ref[0] + y0_ref[...], s1_ref[0] + y1_ref[...]],
                        axis=-1)
    h = h * dinv + b_ref[...][None, :]
    mean = jnp.mean(h, axis=1, keepdims=True)
    var = jnp.mean((h - mean) ** 2, axis=1, keepdims=True)
    h = (h - mean) * lax.rsqrt(var + 1e-5)
    h = h * g_ref[...][None, :] + bb_ref[...][None, :]
    o_ref[...] = 0.5 * h * (1.0 + lax.erf(h * 0.7071067811865476))


def kernel(x, edge_index, W, b, ln_gamma, ln_beta):
    N, D = x.shape
    DO = W.shape[0]
    E = edge_index.shape[1]
    DH = DO // 2

    NP = ((N + 1 + 2047) // 2048) * 2048
    EP = ((E + 4095) // 4096) * 4096
    CA = EP // (2 * NS * CH)
    CB = EP // (NS * CH)

    src = edge_index[0].astype(jnp.int32)
    dst = edge_index[1].astype(jnp.int32)
    pad = EP - E
    src_p = jnp.concatenate([src, jnp.zeros((pad,), jnp.int32)])
    dst_p = jnp.concatenate([dst, jnp.full((pad,), N, jnp.int32)])
    dst_a = dst_p.reshape(2 * NS, CA, CH)
    src_b = src_p.reshape(NS, CB, CH)
    dst_b = dst_p.reshape(NS, CB, CH)

    DW = DH
    ones1 = jnp.ones((CH, DW), jnp.float32)
    zerosd = jnp.zeros((NP, DW), jnp.float32)
    zerosb = jnp.zeros((NP, DH), jnp.float32)

    mesh = plsc.VectorSubcoreMesh(core_axis_name="c", subcore_axis_name="s")

    deg_call = functools.partial(pl.kernel,
        out_type=jax.ShapeDtypeStruct((NC, NP, DW), jnp.float32),
        mesh=mesh,
        scratch_types=[
            pltpu.VMEM((CA, CH), jnp.int32),
            pltpu.VMEM((CH, DW), jnp.float32),
            pltpu.VMEM_SHARED((NP, DW), jnp.float32),
        ])(functools.partial(_deg_body, NP, CA))
    deg = deg_call(dst_a, ones1, zerosd)
    deg0 = deg[0]
    deg1 = deg[1]

    R = 400
    grid = (N // R,)
    dspec = pl.BlockSpec((R, DW), lambda i: (i, 0))
    y = pl.pallas_call(
        _mm_body,
        grid=grid,
        in_specs=[
            pl.BlockSpec((R, D), lambda i: (i, 0)),
            pl.BlockSpec((DO, D), lambda i: (0, 0)),
            dspec, dspec,
        ],
        out_specs=[pl.BlockSpec((R, DH), lambda i: (i, 0)),
                   pl.BlockSpec((R, DH), lambda i: (i, 0))],
        out_shape=[jax.ShapeDtypeStruct((N, DH), jnp.float32),
                   jax.ShapeDtypeStruct((N, DH), jnp.float32)],
    )(x, W, deg0, deg1)
    y0, y1 = y

    agg_call = functools.partial(pl.kernel,
        out_type=jax.ShapeDtypeStruct((NC, NP, DH), jnp.float32),
        mesh=mesh,
        scratch_types=[
            pltpu.VMEM((CB, CH), jnp.int32),
            pltpu.VMEM((2, CH), jnp.int32),
            pltpu.VMEM((CH, DH), jnp.float32),
            pltpu.VMEM((CH, DH), jnp.float32),
            pltpu.VMEM_SHARED((NP, DH), jnp.float32),
            pltpu.SemaphoreType.DMA,
            pltpu.SemaphoreType.DMA,
            pltpu.SemaphoreType.DMA,
            pltpu.SemaphoreType.DMA,
        ])(functools.partial(_agg_body, NP, CB))
    s_parts = agg_call(y0, y1, src_b, dst_b, zerosb)

    sspec0 = pl.BlockSpec((1, R, DH), lambda i: (0, i, 0))
    sspec1 = pl.BlockSpec((1, R, DH), lambda i: (1, i, 0))
    vspec = pl.BlockSpec((DO,), lambda i: (0,))
    out = pl.pallas_call(
        _fin_body,
        grid=grid,
        in_specs=[
            sspec0, sspec1,
            pl.BlockSpec((R, DH), lambda i: (i, 0)),
            pl.BlockSpec((R, DH), lambda i: (i, 0)),
            dspec, dspec,
            vspec, vspec, vspec,
        ],
        out_specs=pl.BlockSpec((R, DO), lambda i: (i, 0)),
        out_shape=jax.ShapeDtypeStruct((N, DO), jnp.float32),
    )(s_parts, s_parts, y0, y1, deg0, deg1, b, ln_gamma, ln_beta)
    return out


# speedup vs baseline: 9.2062x; 1.0426x over previous
import functools

import jax
import jax.numpy as jnp
from jax import lax
from jax.experimental import pallas as pl
from jax.experimental.pallas import tpu as pltpu
from jax.experimental.pallas import tpu_sc as plsc

NC = 2
NS = 16
CH = 128


def _deg_body(np_, ca, dst_hbm, ones_hbm, zeros_hbm, out_hbm, idx_v, ones_v, acc_sh):
    c = lax.axis_index("c")
    s = lax.axis_index("s")
    wid = s * NC + c
    rt = np_ // NS
    pltpu.sync_copy(zeros_hbm.at[pl.ds(s * rt, rt)], acc_sh.at[pl.ds(s * rt, rt)])
    pltpu.sync_copy(ones_hbm, ones_v)
    pltpu.sync_copy(dst_hbm.at[wid], idx_v)
    plsc.subcore_barrier()

    @pl.loop(0, ca)
    def _(j):
        pltpu.sync_copy(ones_v, acc_sh.at[idx_v.at[j]], add=True)

    plsc.subcore_barrier()
    pltpu.sync_copy(acc_sh.at[pl.ds(s * rt, rt)], out_hbm.at[c, pl.ds(s * rt, rt)])


def _agg_body(np_, cb, y0_hbm, y1_hbm, src_hbm, dst_hbm, zeros_hbm, out_hbm,
              dst_v, si, buf0, buf1, acc_sh, semS0, semS1, semG0, semG1,
              semW0, semW1):
    c = lax.axis_index("c")
    s = lax.axis_index("s")
    rt = np_ // NS
    pltpu.sync_copy(zeros_hbm.at[pl.ds(s * rt, rt)], acc_sh.at[pl.ds(s * rt, rt)])
    pltpu.sync_copy(dst_hbm.at[s], dst_v)
    plsc.subcore_barrier()

    def run(y_hbm):
        pltpu.async_copy(src_hbm.at[s, 0], si.at[0], semS0)
        pltpu.async_copy(src_hbm.at[s, 1], si.at[1], semS1)
        pltpu.make_async_copy(src_hbm.at[s, 0], si.at[0], semS0).wait()
        pltpu.async_copy(y_hbm.at[si.at[0]], buf0, semG0)

        @pl.loop(0, cb, step=2)
        def _(j):
            @pl.when(j > 0)
            def _():
                pltpu.make_async_copy(buf1, acc_sh.at[dst_v.at[j - 1]],
                                      semW1).wait()

            pltpu.make_async_copy(src_hbm.at[s, j + 1], si.at[1], semS1).wait()
            pltpu.async_copy(y_hbm.at[si.at[1]], buf1, semG1)

            pltpu.make_async_copy(y_hbm.at[si.at[0]], buf0, semG0).wait()

            @pl.when(j + 2 < cb)
            def _():
                pltpu.async_copy(src_hbm.at[s, j + 2], si.at[0], semS0)

            pltpu.async_copy(buf0, acc_sh.at[dst_v.at[j]], semW0,
                             add=True)

            pltpu.make_async_copy(y_hbm.at[si.at[1]], buf1, semG1).wait()

            @pl.when(j + 3 < cb)
            def _():
                pltpu.async_copy(src_hbm.at[s, j + 3], si.at[1], semS1)

            pltpu.async_copy(buf1, acc_sh.at[dst_v.at[j + 1]], semW1,
                             add=True)

            @pl.when(j + 2 < cb)
            def _():
                pltpu.make_async_copy(buf0, acc_sh.at[dst_v.at[j]],
                                      semW0).wait()
                pltpu.make_async_copy(src_hbm.at[s, j + 2], si.at[0],
                                      semS0).wait()
                pltpu.async_copy(y_hbm.at[si.at[0]], buf0, semG0)

        pltpu.make_async_copy(buf0, acc_sh.at[dst_v.at[cb - 2]], semW0).wait()
        pltpu.make_async_copy(buf1, acc_sh.at[dst_v.at[cb - 1]], semW1).wait()

    @pl.when(c == 0)
    def _():
        run(y0_hbm)

    @pl.when(c == 1)
    def _():
        run(y1_hbm)

    plsc.subcore_barrier()
    pltpu.sync_copy(acc_sh.at[pl.ds(s * rt, rt)], out_hbm.at[c, pl.ds(s * rt, rt)])


def _mm_body(x_ref, w_ref, d0_ref, d1_ref, y0_ref, y1_ref):
    xw = lax.dot_general(x_ref[...], w_ref[...], (((1,), (1,)), ((), ())),
                         preferred_element_type=jnp.float32)
    dinv = lax.rsqrt(d0_ref[:, :1] + d1_ref[:, :1] + 1.0)
    y = xw * dinv
    dh = y.shape[1] // 2
    y0_ref[...] = y[:, :dh]
    y1_ref[...] = y[:, dh:]


def _fin_body(s0_ref, s1_ref, y0_ref, y1_ref, d0_ref, d1_ref, b_ref, g_ref,
              bb_ref, o_ref):
    dinv = lax.rsqrt(d0_ref[:, :1] + d1_ref[:, :1] + 1.0)
    h = jnp.concatenate([s0_ref[0] + y0_ref[...], s1_ref[0] + y1_ref[...]],
                        axis=-1)
    h = h * dinv + b_ref[...][None, :]
    mean = jnp.mean(h, axis=1, keepdims=True)
    var = jnp.mean((h - mean) ** 2, axis=1, keepdims=True)
    h = (h - mean) * lax.rsqrt(var + 1e-5)
    h = h * g_ref[...][None, :] + bb_ref[...][None, :]
    o_ref[...] = 0.5 * h * (1.0 + lax.erf(h * 0.7071067811865476))


def kernel(x, edge_index, W, b, ln_gamma, ln_beta):
    N, D = x.shape
    DO = W.shape[0]
    E = edge_index.shape[1]
    DH = DO // 2

    NP = ((N + 1 + 2047) // 2048) * 2048
    EP = ((E + 4095) // 4096) * 4096
    CA = EP // (2 * NS * CH)
    CB = EP // (NS * CH)

    src = edge_index[0].astype(jnp.int32)
    dst = edge_index[1].astype(jnp.int32)
    pad = EP - E
    src_p = jnp.concatenate([src, jnp.zeros((pad,), jnp.int32)])
    dst_p = jnp.concatenate([dst, jnp.full((pad,), N, jnp.int32)])
    dst_a = dst_p.reshape(2 * NS, CA, CH)
    src_b = src_p.reshape(NS, CB, CH)
    dst_b = dst_p.reshape(NS, CB, CH)

    DW = DH
    ones1 = jnp.ones((CH, DW), jnp.float32)
    zerosd = jnp.zeros((NP, DW), jnp.float32)
    zerosb = jnp.zeros((NP, DH), jnp.float32)

    mesh = plsc.VectorSubcoreMesh(core_axis_name="c", subcore_axis_name="s")

    deg_call = functools.partial(pl.kernel,
        out_type=jax.ShapeDtypeStruct((NC, NP, DW), jnp.float32),
        mesh=mesh,
        scratch_types=[
            pltpu.VMEM((CA, CH), jnp.int32),
            pltpu.VMEM((CH, DW), jnp.float32),
            pltpu.VMEM_SHARED((NP, DW), jnp.float32),
        ])(functools.partial(_deg_body, NP, CA))
    deg = deg_call(dst_a, ones1, zerosd)
    deg0 = deg[0]
    deg1 = deg[1]

    R = 400
    grid = (N // R,)
    dspec = pl.BlockSpec((R, DW), lambda i: (i, 0))
    y = pl.pallas_call(
        _mm_body,
        grid=grid,
        in_specs=[
            pl.BlockSpec((R, D), lambda i: (i, 0)),
            pl.BlockSpec((DO, D), lambda i: (0, 0)),
            dspec, dspec,
        ],
        out_specs=[pl.BlockSpec((R, DH), lambda i: (i, 0)),
                   pl.BlockSpec((R, DH), lambda i: (i, 0))],
        out_shape=[jax.ShapeDtypeStruct((N, DH), jnp.float32),
                   jax.ShapeDtypeStruct((N, DH), jnp.float32)],
    )(x, W, deg0, deg1)
    y0, y1 = y

    agg_call = functools.partial(pl.kernel,
        out_type=jax.ShapeDtypeStruct((NC, NP, DH), jnp.float32),
        mesh=mesh,
        scratch_types=[
            pltpu.VMEM((CB, CH), jnp.int32),
            pltpu.VMEM((2, CH), jnp.int32),
            pltpu.VMEM((CH, DH), jnp.float32),
            pltpu.VMEM((CH, DH), jnp.float32),
            pltpu.VMEM_SHARED((NP, DH), jnp.float32),
            pltpu.SemaphoreType.DMA,
            pltpu.SemaphoreType.DMA,
            pltpu.SemaphoreType.DMA,
            pltpu.SemaphoreType.DMA,
            pltpu.SemaphoreType.DMA,
            pltpu.SemaphoreType.DMA,
        ])(functools.partial(_agg_body, NP, CB))
    s_parts = agg_call(y0, y1, src_b, dst_b, zerosb)

    sspec0 = pl.BlockSpec((1, R, DH), lambda i: (0, i, 0))
    sspec1 = pl.BlockSpec((1, R, DH), lambda i: (1, i, 0))
    vspec = pl.BlockSpec((DO,), lambda i: (0,))
    out = pl.pallas_call(
        _fin_body,
        grid=grid,
        in_specs=[
            sspec0, sspec1,
            pl.BlockSpec((R, DH), lambda i: (i, 0)),
            pl.BlockSpec((R, DH), lambda i: (i, 0)),
            dspec, dspec,
            vspec, vspec, vspec,
        ],
        out_specs=pl.BlockSpec((R, DO), lambda i: (i, 0)),
        out_shape=jax.ShapeDtypeStruct((N, DO), jnp.float32),
    )(s_parts, s_parts, y0, y1, deg0, deg1, b, ln_gamma, ln_beta)
    return out


# speedup vs baseline: 9.4088x; 1.0220x over previous
import functools

import jax
import jax.numpy as jnp
from jax import lax
from jax.experimental import pallas as pl
from jax.experimental.pallas import tpu as pltpu
from jax.experimental.pallas import tpu_sc as plsc

NC = 2
NS = 16
CH = 128


def _deg_body(np_, ca, dst_hbm, ones_hbm, zeros_hbm, out_hbm, idx_v, ones_v, acc_sh, semD):
    c = lax.axis_index("c")
    s = lax.axis_index("s")
    wid = s * NC + c
    rt = np_ // NS
    pltpu.sync_copy(zeros_hbm.at[pl.ds(s * rt, rt)], acc_sh.at[pl.ds(s * rt, rt)])
    pltpu.sync_copy(ones_hbm, ones_v)
    pltpu.sync_copy(dst_hbm.at[wid], idx_v)
    plsc.subcore_barrier()

    @pl.loop(0, ca)
    def _(j):
        pltpu.async_copy(ones_v, acc_sh.at[idx_v.at[j]], semD, add=True)

    @pl.loop(0, ca)
    def _(j):
        pltpu.make_async_copy(ones_v, acc_sh.at[idx_v.at[j]], semD).wait()

    plsc.subcore_barrier()
    pltpu.sync_copy(acc_sh.at[pl.ds(s * rt, rt)], out_hbm.at[c, pl.ds(s * rt, rt)])


def _agg_body(np_, cb, y0_hbm, y1_hbm, src_hbm, dst_hbm, zeros_hbm, out_hbm,
              dst_v, si, buf0, buf1, acc_sh, semS0, semS1, semG0, semG1,
              semW0, semW1):
    c = lax.axis_index("c")
    s = lax.axis_index("s")
    rt = np_ // NS
    pltpu.sync_copy(zeros_hbm.at[pl.ds(s * rt, rt)], acc_sh.at[pl.ds(s * rt, rt)])
    pltpu.sync_copy(dst_hbm.at[s], dst_v)
    plsc.subcore_barrier()

    def run(y_hbm):
        pltpu.async_copy(src_hbm.at[s, 0], si.at[0], semS0)
        pltpu.async_copy(src_hbm.at[s, 1], si.at[1], semS1)
        pltpu.make_async_copy(src_hbm.at[s, 0], si.at[0], semS0).wait()
        pltpu.async_copy(y_hbm.at[si.at[0]], buf0, semG0)

        @pl.loop(0, cb, step=2)
        def _(j):
            @pl.when(j > 0)
            def _():
                pltpu.make_async_copy(buf1, acc_sh.at[dst_v.at[j - 1]],
                                      semW1).wait()

            pltpu.make_async_copy(src_hbm.at[s, j + 1], si.at[1], semS1).wait()
            pltpu.async_copy(y_hbm.at[si.at[1]], buf1, semG1)

            pltpu.make_async_copy(y_hbm.at[si.at[0]], buf0, semG0).wait()

            @pl.when(j + 2 < cb)
            def _():
                pltpu.async_copy(src_hbm.at[s, j + 2], si.at[0], semS0)

            pltpu.async_copy(buf0, acc_sh.at[dst_v.at[j]], semW0,
                             add=True)

            pltpu.make_async_copy(y_hbm.at[si.at[1]], buf1, semG1).wait()

            @pl.when(j + 3 < cb)
            def _():
                pltpu.async_copy(src_hbm.at[s, j + 3], si.at[1], semS1)

            pltpu.async_copy(buf1, acc_sh.at[dst_v.at[j + 1]], semW1,
                             add=True)

            @pl.when(j + 2 < cb)
            def _():
                pltpu.make_async_copy(buf0, acc_sh.at[dst_v.at[j]],
                                      semW0).wait()
                pltpu.make_async_copy(src_hbm.at[s, j + 2], si.at[0],
                                      semS0).wait()
                pltpu.async_copy(y_hbm.at[si.at[0]], buf0, semG0)

        pltpu.make_async_copy(buf0, acc_sh.at[dst_v.at[cb - 2]], semW0).wait()
        pltpu.make_async_copy(buf1, acc_sh.at[dst_v.at[cb - 1]], semW1).wait()

    @pl.when(c == 0)
    def _():
        run(y0_hbm)

    @pl.when(c == 1)
    def _():
        run(y1_hbm)

    plsc.subcore_barrier()
    pltpu.sync_copy(acc_sh.at[pl.ds(s * rt, rt)], out_hbm.at[c, pl.ds(s * rt, rt)])


def _mm_body(x_ref, w_ref, d_ref, y0_ref, y1_ref):
    xw = lax.dot_general(x_ref[...], w_ref[...], (((1,), (1,)), ((), ())),
                         preferred_element_type=jnp.float32)
    dinv = lax.rsqrt(d_ref[...] + 1.0)
    y = xw * dinv
    dh = y.shape[1] // 2
    y0_ref[...] = y[:, :dh]
    y1_ref[...] = y[:, dh:]


def _fin_body(s0_ref, s1_ref, y0_ref, y1_ref, d_ref, b_ref, g_ref,
              bb_ref, o_ref):
    dinv = lax.rsqrt(d_ref[...] + 1.0)
    h = jnp.concatenate([s0_ref[0] + y0_ref[...], s1_ref[0] + y1_ref[...]],
                        axis=-1)
    h = h * dinv + b_ref[...][None, :]
    mean = jnp.mean(h, axis=1, keepdims=True)
    var = jnp.mean((h - mean) ** 2, axis=1, keepdims=True)
    h = (h - mean) * lax.rsqrt(var + 1e-5)
    h = h * g_ref[...][None, :] + bb_ref[...][None, :]
    o_ref[...] = 0.5 * h * (1.0 + lax.erf(h * 0.7071067811865476))


def kernel(x, edge_index, W, b, ln_gamma, ln_beta):
    N, D = x.shape
    DO = W.shape[0]
    E = edge_index.shape[1]
    DH = DO // 2

    NP = ((N + 1 + 2047) // 2048) * 2048
    EP = ((E + 4095) // 4096) * 4096
    CA = EP // (2 * NS * CH)
    CB = EP // (NS * CH)

    src = edge_index[0].astype(jnp.int32)
    dst = edge_index[1].astype(jnp.int32)
    pad = EP - E
    src_p = jnp.concatenate([src, jnp.zeros((pad,), jnp.int32)])
    dst_p = jnp.concatenate([dst, jnp.full((pad,), N, jnp.int32)])
    dst_a = dst_p.reshape(2 * NS, CA, CH)
    src_b = src_p.reshape(NS, CB, CH)
    dst_b = dst_p.reshape(NS, CB, CH)

    DW = DH
    ones1 = jnp.ones((CH, DW), jnp.float32)
    zerosd = jnp.zeros((NP, DW), jnp.float32)
    zerosb = jnp.zeros((NP, DH), jnp.float32)

    mesh = plsc.VectorSubcoreMesh(core_axis_name="c", subcore_axis_name="s")

    deg_call = functools.partial(pl.kernel,
        out_type=jax.ShapeDtypeStruct((NC, NP, DW), jnp.float32),
        mesh=mesh,
        scratch_types=[
            pltpu.VMEM((CA, CH), jnp.int32),
            pltpu.VMEM((CH, DW), jnp.float32),
            pltpu.VMEM_SHARED((NP, DW), jnp.float32),
            pltpu.SemaphoreType.DMA,
        ])(functools.partial(_deg_body, NP, CA))
    deg = deg_call(dst_a, ones1, zerosd)
    degc = deg[0, :, :1] + deg[1, :, :1]

    R = 400
    grid = (N // R,)
    dspec = pl.BlockSpec((R, 1), lambda i: (i, 0))
    y = pl.pallas_call(
        _mm_body,
        grid=grid,
        in_specs=[
            pl.BlockSpec((R, D), lambda i: (i, 0)),
            pl.BlockSpec((DO, D), lambda i: (0, 0)),
            dspec,
        ],
        out_specs=[pl.BlockSpec((R, DH), lambda i: (i, 0)),
                   pl.BlockSpec((R, DH), lambda i: (i, 0))],
        out_shape=[jax.ShapeDtypeStruct((N, DH), jnp.float32),
                   jax.ShapeDtypeStruct((N, DH), jnp.float32)],
    )(x, W, degc)
    y0, y1 = y

    agg_call = functools.partial(pl.kernel,
        out_type=jax.ShapeDtypeStruct((NC, NP, DH), jnp.float32),
        mesh=mesh,
        scratch_types=[
            pltpu.VMEM((CB, CH), jnp.int32),
            pltpu.VMEM((2, CH), jnp.int32),
            pltpu.VMEM((CH, DH), jnp.float32),
            pltpu.VMEM((CH, DH), jnp.float32),
            pltpu.VMEM_SHARED((NP, DH), jnp.float32),
            pltpu.SemaphoreType.DMA,
            pltpu.SemaphoreType.DMA,
            pltpu.SemaphoreType.DMA,
            pltpu.SemaphoreType.DMA,
            pltpu.SemaphoreType.DMA,
            pltpu.SemaphoreType.DMA,
        ])(functools.partial(_agg_body, NP, CB))
    s_parts = agg_call(y0, y1, src_b, dst_b, zerosb)

    sspec0 = pl.BlockSpec((1, R, DH), lambda i: (0, i, 0))
    sspec1 = pl.BlockSpec((1, R, DH), lambda i: (1, i, 0))
    vspec = pl.BlockSpec((DO,), lambda i: (0,))
    out = pl.pallas_call(
        _fin_body,
        grid=grid,
        in_specs=[
            sspec0, sspec1,
            pl.BlockSpec((R, DH), lambda i: (i, 0)),
            pl.BlockSpec((R, DH), lambda i: (i, 0)),
            dspec,
            vspec, vspec, vspec,
        ],
        out_specs=pl.BlockSpec((R, DO), lambda i: (i, 0)),
        out_shape=jax.ShapeDtypeStruct((N, DO), jnp.float32),
    )(s_parts, s_parts, y0, y1, degc, b, ln_gamma, ln_beta)
    return out
